# Initial kernel scaffold; baseline (speedup 1.0000x reference)
import jax
import jax.numpy as jnp
from jax.experimental import pallas as pl


def kernel(u, i, i_c, hist_i, hist_c, sl, item_table, cat_table, item_bias, att_w1, att_b1, att_w2, att_b2, att_w3, att_b3, att_w4, att_b4, bn_gamma, bn_beta, fc1_w, fc1_b, fc2_w, fc2_b, fc3_w, fc3_b):
    raise NotImplementedError("write your pallas kernel here")


# speedup vs baseline: 3.2130x; 3.2130x over previous
import functools

import jax
import jax.numpy as jnp
from jax import lax
from jax.experimental import pallas as pl
from jax.experimental.pallas import tpu as pltpu
from jax.experimental.pallas import tpu_sc as plsc

B = 4096
T = 50
E = 64
H = 2 * E
NC = 2
NS = 16
NW = NC * NS
BT = B * T
HPW = BT // NW
CH = 128
NCH = HPW // CH
QPW = B // NW


def _sc_gather_body(hist_i2, hist_c2, i2, ic2, item_t, cat_t, bias_t,
                    hi_out, hc_out, qi_out, qc_out, bg_out,
                    idx_i, idx_c, idx1, rows_i, rows_c, bias_v,
                    sem_a, sem_b, sem_c):
    wid = lax.axis_index("s") * NC + lax.axis_index("c")

    pltpu.sync_copy(i2.at[wid], idx1)
    ha = pltpu.async_copy(item_t.at[idx1.at[0]], rows_i, sem_a)
    hb = pltpu.async_copy(bias_t.at[idx1.at[0]], bias_v, sem_c)
    hb.wait()
    pltpu.sync_copy(bias_v, bg_out.at[pl.ds(wid * QPW, QPW)])
    ha.wait()
    pltpu.sync_copy(rows_i, qi_out.at[pl.ds(wid * QPW, QPW), :])
    pltpu.sync_copy(ic2.at[wid], idx1)
    pltpu.async_copy(cat_t.at[idx1.at[0]], rows_c, sem_b).wait()
    pltpu.sync_copy(rows_c, qc_out.at[pl.ds(wid * QPW, QPW), :])

    pltpu.sync_copy(hist_i2.at[wid], idx_i)
    pltpu.sync_copy(hist_c2.at[wid], idx_c)
    base = wid * HPW

    def chunk(c, carry):
        ga = pltpu.async_copy(item_t.at[idx_i.at[c]], rows_i, sem_a)
        gb = pltpu.async_copy(cat_t.at[idx_c.at[c]], rows_c, sem_b)
        ga.wait()
        pltpu.sync_copy(rows_i, hi_out.at[pl.ds(base + c * CH, CH), :])
        gb.wait()
        pltpu.sync_copy(rows_c, hc_out.at[pl.ds(base + c * CH, CH), :])
        return carry

    lax.fori_loop(0, NCH, chunk, 0)


def _sc_gather(hist_i, hist_c, i, i_c, item_table, cat_table, item_bias):
    hist_i2 = hist_i.reshape(NW, NCH, CH)
    hist_c2 = hist_c.reshape(NW, NCH, CH)
    i2 = i.reshape(NW, 1, QPW)
    ic2 = i_c.reshape(NW, 1, QPW)
    mesh = plsc.VectorSubcoreMesh(core_axis_name="c", subcore_axis_name="s")
    f = pl.kernel(
        _sc_gather_body,
        out_type=(
            jax.ShapeDtypeStruct((BT, E), jnp.float32),
            jax.ShapeDtypeStruct((BT, E), jnp.float32),
            jax.ShapeDtypeStruct((B, E), jnp.float32),
            jax.ShapeDtypeStruct((B, E), jnp.float32),
            jax.ShapeDtypeStruct((B,), jnp.float32),
        ),
        mesh=mesh,
        compiler_params=pltpu.CompilerParams(use_tc_tiling_on_sc=False),
        scratch_types=[
            pltpu.VMEM((NCH, CH), jnp.int32),
            pltpu.VMEM((NCH, CH), jnp.int32),
            pltpu.VMEM((1, QPW), jnp.int32),
            pltpu.VMEM((CH, E), jnp.float32),
            pltpu.VMEM((CH, E), jnp.float32),
            pltpu.VMEM((QPW,), jnp.float32),
            pltpu.SemaphoreType.DMA,
            pltpu.SemaphoreType.DMA,
            pltpu.SemaphoreType.DMA,
        ],
    )
    return f(hist_i2, hist_c2, i2, ic2, item_table, cat_table, item_bias)


BB = 128
NB = B // BB


def _tc_body(hi_ref, hc_ref, qi_ref, qc_ref, sl_ref, bias_ref,
             w_all_ref, cq_ref, b1_ref, w2_ref, b2_ref, w3_ref,
             wp_ref, wq_ref, bfin_ref, out_ref):
    hi = hi_ref[:]
    hc = hc_ref[:]
    qi = qi_ref[:]
    qc = qc_ref[:]

    q_cat = jnp.concatenate([qi, qc], axis=1)
    q16 = jnp.dot(q_cat, cq_ref[:], preferred_element_type=jnp.float32)
    q16 = q16 + b1_ref[:]

    qi3 = jnp.broadcast_to(qi[:, None, :], (BB, T, E)).reshape(BB * T, E)
    qc3 = jnp.broadcast_to(qc[:, None, :], (BB, T, E)).reshape(BB * T, E)
    x = jnp.concatenate([hi, hc, qi3 * hi, qc3 * hc], axis=1)
    z = jnp.dot(x, w_all_ref[:], preferred_element_type=jnp.float32)
    z1 = jax.nn.sigmoid(z.reshape(BB, T, 16) + q16[:, None, :])
    z2 = jax.nn.sigmoid(
        jnp.dot(z1.reshape(BB * T, 16), w2_ref[:],
                preferred_element_type=jnp.float32) + b2_ref[:])
    s = jnp.dot(z2, w3_ref[:], preferred_element_type=jnp.float32)
    s = s.reshape(BB, T)

    sl = sl_ref[:]
    mask = lax.broadcasted_iota(jnp.int32, (BB, T), 1) < sl
    s = jnp.where(mask, s, -2.0 ** 32 + 1) * (1.0 / (H ** 0.5))
    s = s - jnp.max(s, axis=1, keepdims=True)
    ex = jnp.exp(s)
    attn = ex / jnp.sum(ex, axis=1, keepdims=True)

    v = jnp.dot(jnp.concatenate([hi, hc], axis=1), wp_ref[:],
                preferred_element_type=jnp.float32)
    pooled = jnp.sum(attn * v.reshape(BB, T), axis=1, keepdims=True)

    out = pooled + jnp.dot(q_cat, wq_ref[:], preferred_element_type=jnp.float32)
    out_ref[:] = out + bias_ref[:] + bfin_ref[0, 0]


def _tc_forward(hi, hc, qi, qc, sl2, bias2, w_all, cq, b1, w2, b2, w3,
                wp, wq, bfin):
    full = lambda shape: pl.BlockSpec(shape, lambda b: (0, 0))
    return pl.pallas_call(
        _tc_body,
        grid=(NB,),
        in_specs=[
            pl.BlockSpec((BB * T, E), lambda b: (b, 0)),
            pl.BlockSpec((BB * T, E), lambda b: (b, 0)),
            pl.BlockSpec((BB, E), lambda b: (b, 0)),
            pl.BlockSpec((BB, E), lambda b: (b, 0)),
            pl.BlockSpec((BB, 1), lambda b: (b, 0)),
            pl.BlockSpec((BB, 1), lambda b: (b, 0)),
            full((4 * E, 16)),
            full((H, 16)),
            full((1, 16)),
            full((16, 8)),
            full((1, 8)),
            full((8, 1)),
            full((H, 1)),
            full((H, 1)),
            full((1, 1)),
        ],
        out_specs=pl.BlockSpec((BB, 1), lambda b: (b, 0)),
        out_shape=jax.ShapeDtypeStruct((B, 1), jnp.float32),
    )(hi, hc, qi, qc, sl2, bias2, w_all, cq, b1, w2, b2, w3, wp, wq, bfin)


def kernel(u, i, i_c, hist_i, hist_c, sl, item_table, cat_table, item_bias,
           att_w1, att_b1, att_w2, att_b2, att_w3, att_b3, att_w4, att_b4,
           bn_gamma, bn_beta, fc1_w, fc1_b, fc2_w, fc2_b, fc3_w, fc3_b):
    del u
    w1q, w1h, w1d, w1m = (att_w1[0:H], att_w1[H:2 * H],
                          att_w1[2 * H:3 * H], att_w1[3 * H:4 * H])
    cq = w1q + w1d
    w_all = jnp.concatenate([w1h - w1d, w1m], axis=0)
    g = fc1_w @ fc2_w @ fc3_w
    scale = bn_gamma / jnp.sqrt(1.0 + 1e-3)
    wfin = scale[:, None] * g
    bfin = (bn_beta @ g + (fc1_b @ fc2_w + fc2_b) @ fc3_w + fc3_b
            + att_b4 @ wfin[0:H])
    wp = att_w4 @ wfin[0:H]
    wq = wfin[H:2 * H]

    hi, hc, qi, qc, bg = _sc_gather(
        hist_i.reshape(-1), hist_c.reshape(-1), i, i_c,
        item_table, cat_table, item_bias)

    out = _tc_forward(
        hi, hc, qi, qc, sl.reshape(B, 1), bg.reshape(B, 1),
        w_all, cq, att_b1.reshape(1, 16), att_w2, att_b2.reshape(1, 8),
        att_w3, wp, wq, bfin.reshape(1, 1))
    return out.reshape(-1)


# speedup vs baseline: 4.6513x; 1.4476x over previous
import functools

import jax
import jax.numpy as jnp
from jax import lax
from jax.experimental import pallas as pl
from jax.experimental.pallas import tpu as pltpu
from jax.experimental.pallas import tpu_sc as plsc

B = 4096
T = 50
E = 64
H = 2 * E
NC = 2
NS = 16
NW = NC * NS
BT = B * T
HPW = BT // NW
CH = 128
NCH = HPW // CH
QPW = B // NW


def _sc_gather_body(hist_i2, hist_c2, i2, ic2, item_t, cat_t, bias_t,
                    hi_out, hc_out, qi_out, qc_out, bg_out,
                    idx_i, idx_c, idx1, rows_i, rows_c, bias_v,
                    sem_a, sem_b, sem_c):
    wid = lax.axis_index("s") * NC + lax.axis_index("c")

    pltpu.sync_copy(i2.at[wid], idx1)
    ha = pltpu.async_copy(item_t.at[idx1.at[0]], rows_i, sem_a)
    hb = pltpu.async_copy(bias_t.at[idx1.at[0]], bias_v, sem_c)
    hb.wait()
    pltpu.sync_copy(bias_v, bg_out.at[pl.ds(wid * QPW, QPW)])
    ha.wait()
    pltpu.sync_copy(rows_i, qi_out.at[pl.ds(wid * QPW, QPW), :])
    pltpu.sync_copy(ic2.at[wid], idx1)
    pltpu.async_copy(cat_t.at[idx1.at[0]], rows_c, sem_b).wait()
    pltpu.sync_copy(rows_c, qc_out.at[pl.ds(wid * QPW, QPW), :])

    pltpu.sync_copy(hist_i2.at[wid], idx_i)
    pltpu.sync_copy(hist_c2.at[wid], idx_c)
    base = wid * HPW

    def chunk(c, carry):
        ga = pltpu.async_copy(item_t.at[idx_i.at[c]], rows_i, sem_a)
        gb = pltpu.async_copy(cat_t.at[idx_c.at[c]], rows_c, sem_b)
        ga.wait()
        pltpu.sync_copy(rows_i, hi_out.at[pl.ds(base + c * CH, CH), :])
        gb.wait()
        pltpu.sync_copy(rows_c, hc_out.at[pl.ds(base + c * CH, CH), :])
        return carry

    lax.fori_loop(0, NCH, chunk, 0)


def _sc_gather(hist_i, hist_c, i, i_c, item_table, cat_table, item_bias):
    hist_i2 = hist_i.reshape(NW, NCH, CH)
    hist_c2 = hist_c.reshape(NW, NCH, CH)
    i2 = i.reshape(NW, 1, QPW)
    ic2 = i_c.reshape(NW, 1, QPW)
    mesh = plsc.VectorSubcoreMesh(core_axis_name="c", subcore_axis_name="s")
    f = pl.kernel(
        _sc_gather_body,
        out_type=(
            jax.ShapeDtypeStruct((BT, E), jnp.float32),
            jax.ShapeDtypeStruct((BT, E), jnp.float32),
            jax.ShapeDtypeStruct((B, E), jnp.float32),
            jax.ShapeDtypeStruct((B, E), jnp.float32),
            jax.ShapeDtypeStruct((B,), jnp.float32),
        ),
        mesh=mesh,
        compiler_params=pltpu.CompilerParams(use_tc_tiling_on_sc=False),
        scratch_types=[
            pltpu.VMEM((NCH, CH), jnp.int32),
            pltpu.VMEM((NCH, CH), jnp.int32),
            pltpu.VMEM((1, QPW), jnp.int32),
            pltpu.VMEM((CH, E), jnp.float32),
            pltpu.VMEM((CH, E), jnp.float32),
            pltpu.VMEM((QPW,), jnp.float32),
            pltpu.SemaphoreType.DMA,
            pltpu.SemaphoreType.DMA,
            pltpu.SemaphoreType.DMA,
        ],
    )
    return f(hist_i2, hist_c2, i2, ic2, item_table, cat_table, item_bias)


BB = 128
NB = B // BB
NZ = 32


def _dot(a, b):
    return jnp.dot(a, b, preferred_element_type=jnp.float32)


def _tc_body(hi_ref, hc_ref, qi_ref, qc_ref, sl_ref, bias_ref,
             wz_hi_ref, wz_hc_ref, wz_mi_ref, wz_mc_ref,
             cqi_ref, cqc_ref, b1c_ref, w2t_ref, b2c_ref, w3c_ref,
             bfin_ref, out_ref):
    hi3 = hi_ref[:]
    hc3 = hc_ref[:]
    qi = qi_ref[:]
    qc = qc_ref[:]

    hi2 = hi3.reshape(T * BB, E)
    hc2 = hc3.reshape(T * BB, E)
    qhi2 = (jnp.broadcast_to(qi[None], (T, BB, E)) * hi3).reshape(T * BB, E)
    qhc2 = (jnp.broadcast_to(qc[None], (T, BB, E)) * hc3).reshape(T * BB, E)

    z = (_dot(hi2, wz_hi_ref[:]) + _dot(hc2, wz_hc_ref[:])
         + _dot(qhi2, wz_mi_ref[:]) + _dot(qhc2, wz_mc_ref[:]))
    zt = z.T

    qz = _dot(qi, cqi_ref[:]) + _dot(qc, cqc_ref[:])
    qzt = qz.T
    q16t = jnp.broadcast_to(qzt[0:16][:, None, :], (16, T, BB)).reshape(
        16, T * BB)

    z1s = jax.nn.sigmoid(zt[0:16] + q16t + b1c_ref[:])
    z2s = jax.nn.sigmoid(_dot(w2t_ref[:], z1s) + b2c_ref[:])
    s = jnp.sum(z2s * w3c_ref[:], axis=0, keepdims=True)
    s = s.reshape(T, BB)
    v = zt[16:17].reshape(T, BB)

    sl = sl_ref[0]
    mask = lax.broadcasted_iota(jnp.int32, (T, BB), 0) < sl
    s = jnp.where(mask, s, -2.0 ** 32 + 1) * (1.0 / (H ** 0.5))
    s = s - jnp.max(s, axis=0, keepdims=True)
    ex = jnp.exp(s)
    attn = ex / jnp.sum(ex, axis=0, keepdims=True)

    pooled = jnp.sum(attn * v, axis=0, keepdims=True)
    out = pooled + qzt[16:17] + bias_ref[0] + bfin_ref[0, 0]
    out_ref[0] = out


def _tc_forward(hi, hc, qi, qc, sl3, bias3, wz_hi, wz_hc, wz_mi, wz_mc,
                cqi, cqc, b1c, w2t, b2c, w3c, bfin):
    full = lambda shape: pl.BlockSpec(shape, lambda b: (0, 0))
    return pl.pallas_call(
        _tc_body,
        grid=(NB,),
        in_specs=[
            pl.BlockSpec((T, BB, E), lambda b: (0, b, 0)),
            pl.BlockSpec((T, BB, E), lambda b: (0, b, 0)),
            pl.BlockSpec((BB, E), lambda b: (b, 0)),
            pl.BlockSpec((BB, E), lambda b: (b, 0)),
            pl.BlockSpec((1, 1, BB), lambda b: (b, 0, 0)),
            pl.BlockSpec((1, 1, BB), lambda b: (b, 0, 0)),
            full((E, NZ)),
            full((E, NZ)),
            full((E, NZ)),
            full((E, NZ)),
            full((E, NZ)),
            full((E, NZ)),
            full((16, 1)),
            full((8, 16)),
            full((8, 1)),
            full((8, 1)),
            full((1, 1)),
        ],
        out_specs=pl.BlockSpec((1, 1, BB), lambda b: (b, 0, 0)),
        out_shape=jax.ShapeDtypeStruct((NB, 1, BB), jnp.float32),
    )(hi, hc, qi, qc, sl3, bias3, wz_hi, wz_hc, wz_mi, wz_mc,
      cqi, cqc, b1c, w2t, b2c, w3c, bfin)


def kernel(u, i, i_c, hist_i, hist_c, sl, item_table, cat_table, item_bias,
           att_w1, att_b1, att_w2, att_b2, att_w3, att_b3, att_w4, att_b4,
           bn_gamma, bn_beta, fc1_w, fc1_b, fc2_w, fc2_b, fc3_w, fc3_b):
    del u
    w1q, w1h, w1d, w1m = (att_w1[0:H], att_w1[H:2 * H],
                          att_w1[2 * H:3 * H], att_w1[3 * H:4 * H])
    cq = w1q + w1d
    a = w1h - w1d
    g = fc1_w @ fc2_w @ fc3_w
    scale = bn_gamma / jnp.sqrt(1.0 + 1e-3)
    wfin = scale[:, None] * g
    bfin = (bn_beta @ g + (fc1_b @ fc2_w + fc2_b) @ fc3_w + fc3_b
            + att_b4 @ wfin[0:H])
    wp = att_w4 @ wfin[0:H]
    wq = wfin[H:2 * H]

    zpad = jnp.zeros((E, NZ - 17), jnp.float32)
    wz_hi = jnp.concatenate([a[0:E], wp[0:E], zpad], axis=1)
    wz_hc = jnp.concatenate([a[E:H], wp[E:H], zpad], axis=1)
    zcol = jnp.zeros((E, 1), jnp.float32)
    wz_mi = jnp.concatenate([w1m[0:E], zcol, zpad], axis=1)
    wz_mc = jnp.concatenate([w1m[E:H], zcol, zpad], axis=1)
    cqi = jnp.concatenate([cq[0:E], wq[0:E], zpad], axis=1)
    cqc = jnp.concatenate([cq[E:H], wq[E:H], zpad], axis=1)

    hi, hc, qi, qc, bg = _sc_gather(
        hist_i.T.reshape(-1), hist_c.T.reshape(-1), i, i_c,
        item_table, cat_table, item_bias)

    out = _tc_forward(
        hi.reshape(T, B, E), hc.reshape(T, B, E), qi, qc,
        sl.reshape(NB, 1, BB), bg.reshape(NB, 1, BB),
        wz_hi, wz_hc, wz_mi, wz_mc, cqi, cqc,
        att_b1.reshape(16, 1), att_w2.T, att_b2.reshape(8, 1),
        att_w3.reshape(8, 1), bfin.reshape(1, 1))
    return out.reshape(-1)


# speedup vs baseline: 4.6558x; 1.0010x over previous
import functools

import jax
import jax.numpy as jnp
from jax import lax
from jax.experimental import pallas as pl
from jax.experimental.pallas import tpu as pltpu
from jax.experimental.pallas import tpu_sc as plsc

B = 4096
T = 50
E = 64
H = 2 * E
NC = 2
NS = 16
NW = NC * NS
BT = B * T
HPW = BT // NW
CH = 128
NCH = HPW // CH
QPW = B // NW


def _sc_gather_body(hist_i2, hist_c2, i2, ic2, item_t, cat_t, bias_t,
                    hi_out, hc_out, qi_out, qc_out, bg_out,
                    idx_i, idx_c, idx1, rows_i, rows_c, bias_v,
                    sem_a, sem_b, sem_c):
    wid = lax.axis_index("s") * NC + lax.axis_index("c")

    pltpu.sync_copy(i2.at[wid], idx1)
    ha = pltpu.async_copy(item_t.at[idx1.at[0]], rows_i, sem_a)
    hb = pltpu.async_copy(bias_t.at[idx1.at[0]], bias_v.at[0], sem_c)
    hb.wait()
    pltpu.sync_copy(bias_v, bg_out.at[wid])
    ha.wait()
    pltpu.sync_copy(rows_i, qi_out.at[pl.ds(wid * QPW, QPW), :])
    pltpu.sync_copy(ic2.at[wid], idx1)
    pltpu.async_copy(cat_t.at[idx1.at[0]], rows_c, sem_b).wait()
    pltpu.sync_copy(rows_c, qc_out.at[pl.ds(wid * QPW, QPW), :])

    pltpu.sync_copy(hist_i2.at[wid], idx_i)
    pltpu.sync_copy(hist_c2.at[wid], idx_c)
    base = wid * HPW

    def chunk(c, carry):
        ga = pltpu.async_copy(item_t.at[idx_i.at[c]], rows_i, sem_a)
        gb = pltpu.async_copy(cat_t.at[idx_c.at[c]], rows_c, sem_b)
        flat = base + c * CH
        tt = flat // B
        bb = flat - tt * B
        ga.wait()
        pltpu.sync_copy(rows_i, hi_out.at[tt, pl.ds(bb, CH), :])
        gb.wait()
        pltpu.sync_copy(rows_c, hc_out.at[tt, pl.ds(bb, CH), :])
        return carry

    lax.fori_loop(0, NCH, chunk, 0)


def _sc_gather(hist_i, hist_c, i, i_c, item_table, cat_table, item_bias):
    hist_i2 = hist_i.reshape(NW, NCH, CH)
    hist_c2 = hist_c.reshape(NW, NCH, CH)
    i2 = i.reshape(NW, 1, QPW)
    ic2 = i_c.reshape(NW, 1, QPW)
    mesh = plsc.VectorSubcoreMesh(core_axis_name="c", subcore_axis_name="s")
    f = pl.kernel(
        _sc_gather_body,
        out_type=(
            jax.ShapeDtypeStruct((T, B, E), jnp.float32),
            jax.ShapeDtypeStruct((T, B, E), jnp.float32),
            jax.ShapeDtypeStruct((B, E), jnp.float32),
            jax.ShapeDtypeStruct((B, E), jnp.float32),
            jax.ShapeDtypeStruct((NB, 1, BB), jnp.float32),
        ),
        mesh=mesh,
        compiler_params=pltpu.CompilerParams(use_tc_tiling_on_sc=False),
        scratch_types=[
            pltpu.VMEM((NCH, CH), jnp.int32),
            pltpu.VMEM((NCH, CH), jnp.int32),
            pltpu.VMEM((1, QPW), jnp.int32),
            pltpu.VMEM((CH, E), jnp.float32),
            pltpu.VMEM((CH, E), jnp.float32),
            pltpu.VMEM((1, QPW), jnp.float32),
            pltpu.SemaphoreType.DMA,
            pltpu.SemaphoreType.DMA,
            pltpu.SemaphoreType.DMA,
        ],
    )
    return f(hist_i2, hist_c2, i2, ic2, item_table, cat_table, item_bias)


BB = 128
NB = B // BB
NZ = 32


def _dot(a, b):
    return jnp.dot(a, b, preferred_element_type=jnp.float32)


def _tc_body(hi_ref, hc_ref, qi_ref, qc_ref, sl_ref, bias_ref,
             wz_hi_ref, wz_hc_ref, wz_mi_ref, wz_mc_ref,
             cqi_ref, cqc_ref, b1c_ref, w2t_ref, b2c_ref, w3c_ref,
             bfin_ref, out_ref):
    hi3 = hi_ref[:]
    hc3 = hc_ref[:]
    qi = qi_ref[:]
    qc = qc_ref[:]

    hi2 = hi3.reshape(T * BB, E)
    hc2 = hc3.reshape(T * BB, E)
    qhi2 = (jnp.broadcast_to(qi[None], (T, BB, E)) * hi3).reshape(T * BB, E)
    qhc2 = (jnp.broadcast_to(qc[None], (T, BB, E)) * hc3).reshape(T * BB, E)

    z = (_dot(hi2, wz_hi_ref[:]) + _dot(hc2, wz_hc_ref[:])
         + _dot(qhi2, wz_mi_ref[:]) + _dot(qhc2, wz_mc_ref[:]))
    zt = z.T

    qz = _dot(qi, cqi_ref[:]) + _dot(qc, cqc_ref[:])
    qzt = qz.T
    q16t = jnp.broadcast_to(qzt[0:16][:, None, :], (16, T, BB)).reshape(
        16, T * BB)

    z1s = jax.nn.sigmoid(zt[0:16] + q16t + b1c_ref[:])
    z2s = jax.nn.sigmoid(_dot(w2t_ref[:], z1s) + b2c_ref[:])
    s = jnp.sum(z2s * w3c_ref[:], axis=0, keepdims=True)
    s = s.reshape(T, BB)
    v = zt[16:17].reshape(T, BB)

    sl = sl_ref[0]
    mask = lax.broadcasted_iota(jnp.int32, (T, BB), 0) < sl
    s = jnp.where(mask, s, -2.0 ** 32 + 1) * (1.0 / (H ** 0.5))
    s = s - jnp.max(s, axis=0, keepdims=True)
    ex = jnp.exp(s)
    attn = ex / jnp.sum(ex, axis=0, keepdims=True)

    pooled = jnp.sum(attn * v, axis=0, keepdims=True)
    out = pooled + qzt[16:17] + bias_ref[0] + bfin_ref[0, 0]
    out_ref[0] = out


def _tc_forward(hi, hc, qi, qc, sl3, bias3, wz_hi, wz_hc, wz_mi, wz_mc,
                cqi, cqc, b1c, w2t, b2c, w3c, bfin):
    full = lambda shape: pl.BlockSpec(shape, lambda b: (0, 0))
    return pl.pallas_call(
        _tc_body,
        grid=(NB,),
        in_specs=[
            pl.BlockSpec((T, BB, E), lambda b: (0, b, 0)),
            pl.BlockSpec((T, BB, E), lambda b: (0, b, 0)),
            pl.BlockSpec((BB, E), lambda b: (b, 0)),
            pl.BlockSpec((BB, E), lambda b: (b, 0)),
            pl.BlockSpec((1, 1, BB), lambda b: (b, 0, 0)),
            pl.BlockSpec((1, 1, BB), lambda b: (b, 0, 0)),
            full((E, NZ)),
            full((E, NZ)),
            full((E, NZ)),
            full((E, NZ)),
            full((E, NZ)),
            full((E, NZ)),
            full((16, 1)),
            full((8, 16)),
            full((8, 1)),
            full((8, 1)),
            full((1, 1)),
        ],
        out_specs=pl.BlockSpec((1, 1, BB), lambda b: (b, 0, 0)),
        out_shape=jax.ShapeDtypeStruct((NB, 1, BB), jnp.float32),
    )(hi, hc, qi, qc, sl3, bias3, wz_hi, wz_hc, wz_mi, wz_mc,
      cqi, cqc, b1c, w2t, b2c, w3c, bfin)


def kernel(u, i, i_c, hist_i, hist_c, sl, item_table, cat_table, item_bias,
           att_w1, att_b1, att_w2, att_b2, att_w3, att_b3, att_w4, att_b4,
           bn_gamma, bn_beta, fc1_w, fc1_b, fc2_w, fc2_b, fc3_w, fc3_b):
    del u
    w1q, w1h, w1d, w1m = (att_w1[0:H], att_w1[H:2 * H],
                          att_w1[2 * H:3 * H], att_w1[3 * H:4 * H])
    cq = w1q + w1d
    a = w1h - w1d
    g = fc1_w @ fc2_w @ fc3_w
    scale = bn_gamma / jnp.sqrt(1.0 + 1e-3)
    wfin = scale[:, None] * g
    bfin = (bn_beta @ g + (fc1_b @ fc2_w + fc2_b) @ fc3_w + fc3_b
            + att_b4 @ wfin[0:H])
    wp = att_w4 @ wfin[0:H]
    wq = wfin[H:2 * H]

    zpad = jnp.zeros((E, NZ - 17), jnp.float32)
    wz_hi = jnp.concatenate([a[0:E], wp[0:E], zpad], axis=1)
    wz_hc = jnp.concatenate([a[E:H], wp[E:H], zpad], axis=1)
    zcol = jnp.zeros((E, 1), jnp.float32)
    wz_mi = jnp.concatenate([w1m[0:E], zcol, zpad], axis=1)
    wz_mc = jnp.concatenate([w1m[E:H], zcol, zpad], axis=1)
    cqi = jnp.concatenate([cq[0:E], wq[0:E], zpad], axis=1)
    cqc = jnp.concatenate([cq[E:H], wq[E:H], zpad], axis=1)

    hi, hc, qi, qc, bg = _sc_gather(
        hist_i.T.reshape(-1), hist_c.T.reshape(-1), i, i_c,
        item_table, cat_table, item_bias)

    out = _tc_forward(
        hi, hc, qi, qc,
        sl.reshape(NB, 1, BB), bg,
        wz_hi, wz_hc, wz_mi, wz_mc, cqi, cqc,
        att_b1.reshape(16, 1), att_w2.T, att_b2.reshape(8, 1),
        att_w3.reshape(8, 1), bfin.reshape(1, 1))
    return out.reshape(-1)


# speedup vs baseline: 7.9298x; 1.7032x over previous
import functools

import jax
import jax.numpy as jnp
from jax import lax
from jax.experimental import pallas as pl
from jax.experimental.pallas import tpu as pltpu
from jax.experimental.pallas import tpu_sc as plsc

B = 4096
T = 50
E = 64
H = 2 * E
NC = 2
NS = 16
NW = NC * NS
BT = B * T
HPW = BT // NW
CH = 128
NCH = HPW // CH
QPW = B // NW


def _sc_gather_body(hist_i2, hist_c2, i2, ic2, item_t, cat_t, bias_t,
                    h_out, q_out, bg_out,
                    idx_i, idx_c, idx1, rows_i, rows_c, bias_v,
                    sem_a, sem_b, sem_c):
    wid = lax.axis_index("s") * NC + lax.axis_index("c")

    pltpu.sync_copy(i2.at[wid], idx1)
    ha = pltpu.async_copy(item_t.at[idx1.at[0]], rows_i, sem_a)
    hb = pltpu.async_copy(bias_t.at[idx1.at[0]], bias_v.at[0], sem_c)
    hb.wait()
    pltpu.sync_copy(bias_v, bg_out.at[wid])
    ha.wait()
    pltpu.sync_copy(rows_i, q_out.at[pl.ds(wid * QPW, QPW), pl.ds(0, E)])
    pltpu.sync_copy(ic2.at[wid], idx1)
    pltpu.async_copy(cat_t.at[idx1.at[0]], rows_c, sem_b).wait()
    pltpu.sync_copy(rows_c, q_out.at[pl.ds(wid * QPW, QPW), pl.ds(E, E)])

    pltpu.sync_copy(hist_i2.at[wid], idx_i)
    pltpu.sync_copy(hist_c2.at[wid], idx_c)
    base = wid * HPW

    def chunk(c, carry):
        ga = pltpu.async_copy(item_t.at[idx_i.at[c]], rows_i, sem_a)
        gb = pltpu.async_copy(cat_t.at[idx_c.at[c]], rows_c, sem_b)
        flat = base + c * CH
        tt = flat // B
        bb = flat - tt * B
        ga.wait()
        pltpu.sync_copy(rows_i, h_out.at[tt, pl.ds(bb, CH), pl.ds(0, E)])
        gb.wait()
        pltpu.sync_copy(rows_c, h_out.at[tt, pl.ds(bb, CH), pl.ds(E, E)])
        return carry

    lax.fori_loop(0, NCH, chunk, 0)


def _sc_gather(hist_i, hist_c, i, i_c, item_table, cat_table, item_bias):
    hist_i2 = hist_i.reshape(NW, NCH, CH)
    hist_c2 = hist_c.reshape(NW, NCH, CH)
    i2 = i.reshape(NW, 1, QPW)
    ic2 = i_c.reshape(NW, 1, QPW)
    mesh = plsc.VectorSubcoreMesh(core_axis_name="c", subcore_axis_name="s")
    f = pl.kernel(
        _sc_gather_body,
        out_type=(
            jax.ShapeDtypeStruct((T, B, H), jnp.float32),
            jax.ShapeDtypeStruct((B, H), jnp.float32),
            jax.ShapeDtypeStruct((NB, 1, BB), jnp.float32),
        ),
        mesh=mesh,
        compiler_params=pltpu.CompilerParams(use_tc_tiling_on_sc=False),
        scratch_types=[
            pltpu.VMEM((NCH, CH), jnp.int32),
            pltpu.VMEM((NCH, CH), jnp.int32),
            pltpu.VMEM((1, QPW), jnp.int32),
            pltpu.VMEM((CH, E), jnp.float32),
            pltpu.VMEM((CH, E), jnp.float32),
            pltpu.VMEM((1, QPW), jnp.float32),
            pltpu.SemaphoreType.DMA,
            pltpu.SemaphoreType.DMA,
            pltpu.SemaphoreType.DMA,
        ],
    )
    return f(hist_i2, hist_c2, i2, ic2, item_table, cat_table, item_bias)


BB = 128
NB = B // BB
NZ = 32


def _dot(a, b):
    return jnp.dot(a, b, preferred_element_type=jnp.float32)


def _tc_body(h_ref, q_ref, sl_ref, bias_ref,
             wz_h_ref, wz_m_ref, cq_ref, b1c_ref, w2t_ref, b2c_ref, w3c_ref,
             bfin_ref, out_ref):
    h3 = h_ref[:]
    q = q_ref[:]

    h2 = h3.reshape(T * BB, H)
    qh2 = (jnp.broadcast_to(q[None], (T, BB, H)) * h3).reshape(T * BB, H)

    z = _dot(h2, wz_h_ref[:]) + _dot(qh2, wz_m_ref[:])
    zt = z.T

    qz = _dot(q, cq_ref[:])
    qzt = qz.T
    q16t = jnp.broadcast_to(qzt[0:16][:, None, :], (16, T, BB)).reshape(
        16, T * BB)

    z1s = jax.nn.sigmoid(zt[0:16] + q16t + b1c_ref[:])
    z2s = jax.nn.sigmoid(_dot(w2t_ref[:], z1s) + b2c_ref[:])
    s = jnp.sum(z2s * w3c_ref[:], axis=0, keepdims=True)
    s = s.reshape(T, BB)
    v = zt[16:17].reshape(T, BB)

    sl = sl_ref[0]
    mask = lax.broadcasted_iota(jnp.int32, (T, BB), 0) < sl
    s = jnp.where(mask, s, -2.0 ** 32 + 1) * (1.0 / (H ** 0.5))
    s = s - jnp.max(s, axis=0, keepdims=True)
    ex = jnp.exp(s)
    attn = ex / jnp.sum(ex, axis=0, keepdims=True)

    pooled = jnp.sum(attn * v, axis=0, keepdims=True)
    out = pooled + qzt[16:17] + bias_ref[0] + bfin_ref[0, 0]
    out_ref[0] = out


def _tc_forward(h, q, sl3, bias3, wz_h, wz_m, cq_ext, b1c, w2t, b2c, w3c,
                bfin):
    full = lambda shape: pl.BlockSpec(shape, lambda b: (0, 0))
    return pl.pallas_call(
        _tc_body,
        grid=(NB,),
        in_specs=[
            pl.BlockSpec((T, BB, H), lambda b: (0, b, 0)),
            pl.BlockSpec((BB, H), lambda b: (b, 0)),
            pl.BlockSpec((1, 1, BB), lambda b: (b, 0, 0)),
            pl.BlockSpec((1, 1, BB), lambda b: (b, 0, 0)),
            full((H, NZ)),
            full((H, NZ)),
            full((H, NZ)),
            full((16, 1)),
            full((8, 16)),
            full((8, 1)),
            full((8, 1)),
            full((1, 1)),
        ],
        out_specs=pl.BlockSpec((1, 1, BB), lambda b: (b, 0, 0)),
        out_shape=jax.ShapeDtypeStruct((NB, 1, BB), jnp.float32),
    )(h, q, sl3, bias3, wz_h, wz_m, cq_ext, b1c, w2t, b2c, w3c, bfin)


def kernel(u, i, i_c, hist_i, hist_c, sl, item_table, cat_table, item_bias,
           att_w1, att_b1, att_w2, att_b2, att_w3, att_b3, att_w4, att_b4,
           bn_gamma, bn_beta, fc1_w, fc1_b, fc2_w, fc2_b, fc3_w, fc3_b):
    del u
    w1q, w1h, w1d, w1m = (att_w1[0:H], att_w1[H:2 * H],
                          att_w1[2 * H:3 * H], att_w1[3 * H:4 * H])
    cq = w1q + w1d
    a = w1h - w1d
    g = fc1_w @ fc2_w @ fc3_w
    scale = bn_gamma / jnp.sqrt(1.0 + 1e-3)
    wfin = scale[:, None] * g
    bfin = (bn_beta @ g + (fc1_b @ fc2_w + fc2_b) @ fc3_w + fc3_b
            + att_b4 @ wfin[0:H])
    wp = att_w4 @ wfin[0:H]
    wq = wfin[H:2 * H]

    zpad = jnp.zeros((H, NZ - 17), jnp.float32)
    zcol = jnp.zeros((H, 1), jnp.float32)
    wz_h = jnp.concatenate([a, wp, zpad], axis=1)
    wz_m = jnp.concatenate([w1m, zcol, zpad], axis=1)
    cq_ext = jnp.concatenate([cq, wq, zpad], axis=1)

    h, q, bg = _sc_gather(
        hist_i.T.reshape(-1), hist_c.T.reshape(-1), i, i_c,
        item_table, cat_table, item_bias)

    out = _tc_forward(
        h, q, sl.reshape(NB, 1, BB), bg,
        wz_h, wz_m, cq_ext,
        att_b1.reshape(16, 1), att_w2.T, att_b2.reshape(8, 1),
        att_w3.reshape(8, 1), bfin.reshape(1, 1))
    return out.reshape(-1)


# speedup vs baseline: 7.9738x; 1.0055x over previous
import functools

import jax
import jax.numpy as jnp
from jax import lax
from jax.experimental import pallas as pl
from jax.experimental.pallas import tpu as pltpu
from jax.experimental.pallas import tpu_sc as plsc

B = 4096
T = 50
E = 64
H = 2 * E
NC = 2
NS = 16
NW = NC * NS
BT = B * T
HPW = BT // NW
CH = 128
NCH = HPW // CH
GK = 5
QPW = B // NW


def _sc_gather_body(hist_i2, hist_c2, i2, ic2, item_t, cat_t, bias_t,
                    h_out, q_out, bg_out,
                    idx_i, idx_c, idx1, rows_i, rows_c, bias_v,
                    sem_a, sem_b, sem_c, sem_d):
    wid = lax.axis_index("s") * NC + lax.axis_index("c")

    pltpu.sync_copy(i2.at[wid], idx1)
    ha = pltpu.async_copy(item_t.at[idx1.at[0]], rows_i.at[pl.ds(0, CH)],
                          sem_a)
    hb = pltpu.async_copy(bias_t.at[idx1.at[0]], bias_v.at[0], sem_c)
    hb.wait()
    pltpu.sync_copy(bias_v, bg_out.at[wid])
    ha.wait()
    pltpu.sync_copy(rows_i.at[pl.ds(0, CH)],
                    q_out.at[pl.ds(wid * QPW, QPW), pl.ds(0, E)])
    pltpu.sync_copy(ic2.at[wid], idx1)
    pltpu.async_copy(cat_t.at[idx1.at[0]], rows_c.at[pl.ds(0, CH)],
                     sem_b).wait()
    pltpu.sync_copy(rows_c.at[pl.ds(0, CH)],
                    q_out.at[pl.ds(wid * QPW, QPW), pl.ds(E, E)])

    pltpu.sync_copy(hist_i2.at[wid], idx_i)
    pltpu.sync_copy(hist_c2.at[wid], idx_c)
    base = wid * HPW

    def group(g, carry):
        c0 = g * GK
        hs = [pltpu.async_copy(item_t.at[idx_i.at[c0 + k]],
                               rows_i.at[pl.ds(k * CH, CH)], sem_a)
              for k in range(GK)]
        gs = [pltpu.async_copy(cat_t.at[idx_c.at[c0 + k]],
                               rows_c.at[pl.ds(k * CH, CH)], sem_b)
              for k in range(GK)]
        outs = []
        for k in range(GK):
            flat = base + (c0 + k) * CH
            tt = flat // B
            bb = flat - tt * B
            hs[k].wait()
            outs.append(pltpu.async_copy(
                rows_i.at[pl.ds(k * CH, CH)],
                h_out.at[tt, pl.ds(bb, CH), pl.ds(0, E)], sem_c))
        for k in range(GK):
            flat = base + (c0 + k) * CH
            tt = flat // B
            bb = flat - tt * B
            gs[k].wait()
            outs.append(pltpu.async_copy(
                rows_c.at[pl.ds(k * CH, CH)],
                h_out.at[tt, pl.ds(bb, CH), pl.ds(E, E)], sem_d))
        for o in outs:
            o.wait()
        return carry

    lax.fori_loop(0, NCH // GK, group, 0)


def _sc_gather(hist_i, hist_c, i, i_c, item_table, cat_table, item_bias):
    hist_i2 = hist_i.reshape(NW, NCH, CH)
    hist_c2 = hist_c.reshape(NW, NCH, CH)
    i2 = i.reshape(NW, 1, QPW)
    ic2 = i_c.reshape(NW, 1, QPW)
    mesh = plsc.VectorSubcoreMesh(core_axis_name="c", subcore_axis_name="s")
    f = pl.kernel(
        _sc_gather_body,
        out_type=(
            jax.ShapeDtypeStruct((T, B, H), jnp.float32),
            jax.ShapeDtypeStruct((B, H), jnp.float32),
            jax.ShapeDtypeStruct((NB, 1, BB), jnp.float32),
        ),
        mesh=mesh,
        compiler_params=pltpu.CompilerParams(use_tc_tiling_on_sc=False),
        scratch_types=[
            pltpu.VMEM((NCH, CH), jnp.int32),
            pltpu.VMEM((NCH, CH), jnp.int32),
            pltpu.VMEM((1, QPW), jnp.int32),
            pltpu.VMEM((GK * CH, E), jnp.float32),
            pltpu.VMEM((GK * CH, E), jnp.float32),
            pltpu.VMEM((1, QPW), jnp.float32),
            pltpu.SemaphoreType.DMA,
            pltpu.SemaphoreType.DMA,
            pltpu.SemaphoreType.DMA,
            pltpu.SemaphoreType.DMA,
        ],
    )
    return f(hist_i2, hist_c2, i2, ic2, item_table, cat_table, item_bias)


BB = 128
NB = B // BB
NZ = 32


def _dot(a, b):
    return jnp.dot(a, b, preferred_element_type=jnp.float32)


def _tc_body(h_ref, q_ref, sl_ref, bias_ref,
             wz_h_ref, wz_m_ref, cq_ref, b1c_ref, w2t_ref, b2c_ref, w3c_ref,
             bfin_ref, out_ref):
    h3 = h_ref[:]
    q = q_ref[:]

    h2 = h3.reshape(T * BB, H)
    qh2 = (jnp.broadcast_to(q[None], (T, BB, H)) * h3).reshape(T * BB, H)

    z = _dot(h2, wz_h_ref[:]) + _dot(qh2, wz_m_ref[:])
    zt = z.T

    qz = _dot(q, cq_ref[:])
    qzt = qz.T
    q16t = jnp.broadcast_to(qzt[0:16][:, None, :], (16, T, BB)).reshape(
        16, T * BB)

    z1s = jax.nn.sigmoid(zt[0:16] + q16t + b1c_ref[:])
    z2s = jax.nn.sigmoid(_dot(w2t_ref[:], z1s) + b2c_ref[:])
    s = jnp.sum(z2s * w3c_ref[:], axis=0, keepdims=True)
    s = s.reshape(T, BB)
    v = zt[16:17].reshape(T, BB)

    sl = sl_ref[0]
    mask = lax.broadcasted_iota(jnp.int32, (T, BB), 0) < sl
    s = jnp.where(mask, s, -2.0 ** 32 + 1) * (1.0 / (H ** 0.5))
    s = s - jnp.max(s, axis=0, keepdims=True)
    ex = jnp.exp(s)
    attn = ex / jnp.sum(ex, axis=0, keepdims=True)

    pooled = jnp.sum(attn * v, axis=0, keepdims=True)
    out = pooled + qzt[16:17] + bias_ref[0] + bfin_ref[0, 0]
    out_ref[0] = out


def _tc_forward(h, q, sl3, bias3, wz_h, wz_m, cq_ext, b1c, w2t, b2c, w3c,
                bfin):
    full = lambda shape: pl.BlockSpec(shape, lambda b: (0, 0))
    return pl.pallas_call(
        _tc_body,
        grid=(NB,),
        in_specs=[
            pl.BlockSpec((T, BB, H), lambda b: (0, b, 0)),
            pl.BlockSpec((BB, H), lambda b: (b, 0)),
            pl.BlockSpec((1, 1, BB), lambda b: (b, 0, 0)),
            pl.BlockSpec((1, 1, BB), lambda b: (b, 0, 0)),
            full((H, NZ)),
            full((H, NZ)),
            full((H, NZ)),
            full((16, 1)),
            full((8, 16)),
            full((8, 1)),
            full((8, 1)),
            full((1, 1)),
        ],
        out_specs=pl.BlockSpec((1, 1, BB), lambda b: (b, 0, 0)),
        out_shape=jax.ShapeDtypeStruct((NB, 1, BB), jnp.float32),
    )(h, q, sl3, bias3, wz_h, wz_m, cq_ext, b1c, w2t, b2c, w3c, bfin)


def kernel(u, i, i_c, hist_i, hist_c, sl, item_table, cat_table, item_bias,
           att_w1, att_b1, att_w2, att_b2, att_w3, att_b3, att_w4, att_b4,
           bn_gamma, bn_beta, fc1_w, fc1_b, fc2_w, fc2_b, fc3_w, fc3_b):
    del u
    w1q, w1h, w1d, w1m = (att_w1[0:H], att_w1[H:2 * H],
                          att_w1[2 * H:3 * H], att_w1[3 * H:4 * H])
    cq = w1q + w1d
    a = w1h - w1d
    g = fc1_w @ fc2_w @ fc3_w
    scale = bn_gamma / jnp.sqrt(1.0 + 1e-3)
    wfin = scale[:, None] * g
    bfin = (bn_beta @ g + (fc1_b @ fc2_w + fc2_b) @ fc3_w + fc3_b
            + att_b4 @ wfin[0:H])
    wp = att_w4 @ wfin[0:H]
    wq = wfin[H:2 * H]

    zpad = jnp.zeros((H, NZ - 17), jnp.float32)
    zcol = jnp.zeros((H, 1), jnp.float32)
    wz_h = jnp.concatenate([a, wp, zpad], axis=1)
    wz_m = jnp.concatenate([w1m, zcol, zpad], axis=1)
    cq_ext = jnp.concatenate([cq, wq, zpad], axis=1)

    h, q, bg = _sc_gather(
        hist_i.T.reshape(-1), hist_c.T.reshape(-1), i, i_c,
        item_table, cat_table, item_bias)

    out = _tc_forward(
        h, q, sl.reshape(NB, 1, BB), bg,
        wz_h, wz_m, cq_ext,
        att_b1.reshape(16, 1), att_w2.T, att_b2.reshape(8, 1),
        att_w3.reshape(8, 1), bfin.reshape(1, 1))
    return out.reshape(-1)


# speedup vs baseline: 7.9996x; 1.0032x over previous
import functools

import jax
import jax.numpy as jnp
from jax import lax
from jax.experimental import pallas as pl
from jax.experimental.pallas import tpu as pltpu
from jax.experimental.pallas import tpu_sc as plsc

B = 4096
T = 50
E = 64
H = 2 * E
EP = E // 2
HP = 2 * EP
NC = 2
NS = 16
NW = NC * NS
BT = B * T
HPW = BT // NW
CH = 128
NCH = HPW // CH
GK = 5
QPW = B // NW


def _sc_gather_body(hist_i2, hist_c2, i2, ic2, item_t, cat_t, bias_t,
                    h_out, q_out, bg_out,
                    idx_i, idx_c, idx1, rows_i, rows_c, bias_v,
                    sem_a, sem_b, sem_c, sem_d):
    wid = lax.axis_index("s") * NC + lax.axis_index("c")

    pltpu.sync_copy(i2.at[wid], idx1)
    ha = pltpu.async_copy(item_t.at[idx1.at[0]], rows_i.at[pl.ds(0, CH)],
                          sem_a)
    hb = pltpu.async_copy(bias_t.at[idx1.at[0]], bias_v.at[0], sem_c)
    hb.wait()
    pltpu.sync_copy(bias_v, bg_out.at[wid])
    ha.wait()
    pltpu.sync_copy(rows_i.at[pl.ds(0, CH)],
                    q_out.at[pl.ds(wid * QPW, QPW), pl.ds(0, E)])
    pltpu.sync_copy(ic2.at[wid], idx1)
    pltpu.async_copy(cat_t.at[idx1.at[0]], rows_c.at[pl.ds(0, CH)],
                     sem_b).wait()
    pltpu.sync_copy(rows_c.at[pl.ds(0, CH)],
                    q_out.at[pl.ds(wid * QPW, QPW), pl.ds(E, E)])

    pltpu.sync_copy(hist_i2.at[wid], idx_i)
    pltpu.sync_copy(hist_c2.at[wid], idx_c)
    base = wid * HPW

    def group(g, carry):
        c0 = g * GK
        hs = [pltpu.async_copy(item_t.at[idx_i.at[c0 + k]],
                               rows_i.at[pl.ds(k * CH, CH)], sem_a)
              for k in range(GK)]
        gs = [pltpu.async_copy(cat_t.at[idx_c.at[c0 + k]],
                               rows_c.at[pl.ds(k * CH, CH)], sem_b)
              for k in range(GK)]
        outs = []
        for k in range(GK):
            flat = base + (c0 + k) * CH
            tt = flat // B
            bb = flat - tt * B
            hs[k].wait()
            outs.append(pltpu.async_copy(
                rows_i.at[pl.ds(k * CH, CH)],
                h_out.at[tt, pl.ds(bb, CH), pl.ds(0, E)], sem_c))
        for k in range(GK):
            flat = base + (c0 + k) * CH
            tt = flat // B
            bb = flat - tt * B
            gs[k].wait()
            outs.append(pltpu.async_copy(
                rows_c.at[pl.ds(k * CH, CH)],
                h_out.at[tt, pl.ds(bb, CH), pl.ds(E, E)], sem_d))
        for o in outs:
            o.wait()
        return carry

    lax.fori_loop(0, NCH // GK, group, 0)


def _sc_gather(hist_i, hist_c, i, i_c, item_table, cat_table, item_bias):
    hist_i2 = hist_i.reshape(NW, NCH, CH)
    hist_c2 = hist_c.reshape(NW, NCH, CH)
    i2 = i.reshape(NW, 1, QPW)
    ic2 = i_c.reshape(NW, 1, QPW)
    mesh = plsc.VectorSubcoreMesh(core_axis_name="c", subcore_axis_name="s")
    f = pl.kernel(
        _sc_gather_body,
        out_type=(
            jax.ShapeDtypeStruct((T, B, H), jnp.float32),
            jax.ShapeDtypeStruct((B, H), jnp.float32),
            jax.ShapeDtypeStruct((NB, 1, BB), jnp.float32),
        ),
        mesh=mesh,
        compiler_params=pltpu.CompilerParams(use_tc_tiling_on_sc=False),
        scratch_types=[
            pltpu.VMEM((NCH, CH), jnp.int32),
            pltpu.VMEM((NCH, CH), jnp.int32),
            pltpu.VMEM((1, QPW), jnp.int32),
            pltpu.VMEM((GK * CH, E), jnp.float32),
            pltpu.VMEM((GK * CH, E), jnp.float32),
            pltpu.VMEM((1, QPW), jnp.float32),
            pltpu.SemaphoreType.DMA,
            pltpu.SemaphoreType.DMA,
            pltpu.SemaphoreType.DMA,
            pltpu.SemaphoreType.DMA,
        ],
    )
    return f(hist_i2, hist_c2, i2, ic2, item_table, cat_table, item_bias)


BB = 128
NB = B // BB
NZ = 32


def _dot(a, b):
    return jnp.dot(a, b, preferred_element_type=jnp.float32)


def _tc_body(h_ref, q_ref, sl_ref, bias_ref,
             wz_h_ref, wz_m_ref, cq_ref, b1c_ref, w2t_ref, b2c_ref, w3c_ref,
             bfin_ref, out_ref):
    h3 = h_ref[:]
    q = q_ref[:]

    h2 = h3.reshape(T * BB, H)
    qh2 = (jnp.broadcast_to(q[None], (T, BB, H)) * h3).reshape(T * BB, H)

    z = _dot(h2, wz_h_ref[:]) + _dot(qh2, wz_m_ref[:])
    zt = z.T

    qz = _dot(q, cq_ref[:])
    qzt = qz.T
    q16t = jnp.broadcast_to(qzt[0:16][:, None, :], (16, T, BB)).reshape(
        16, T * BB)

    z1s = jax.nn.sigmoid(zt[0:16] + q16t + b1c_ref[:])
    z2s = jax.nn.sigmoid(_dot(w2t_ref[:], z1s) + b2c_ref[:])
    s = jnp.sum(z2s * w3c_ref[:], axis=0, keepdims=True)
    s = s.reshape(T, BB)
    v = zt[16:17].reshape(T, BB)

    sl = sl_ref[0]
    mask = lax.broadcasted_iota(jnp.int32, (T, BB), 0) < sl
    s = jnp.where(mask, s, -2.0 ** 32 + 1) * (1.0 / (H ** 0.5))
    s = s - jnp.max(s, axis=0, keepdims=True)
    ex = jnp.exp(s)
    attn = ex / jnp.sum(ex, axis=0, keepdims=True)

    pooled = jnp.sum(attn * v, axis=0, keepdims=True)
    out = pooled + qzt[16:17] + bias_ref[0] + bfin_ref[0, 0]
    out_ref[0] = out


def _tc_forward(h, q, sl3, bias3, wz_h, wz_m, cq_ext, b1c, w2t, b2c, w3c,
                bfin):
    full = lambda shape: pl.BlockSpec(shape, lambda b: (0, 0))
    return pl.pallas_call(
        _tc_body,
        grid=(NB,),
        in_specs=[
            pl.BlockSpec((T, BB, H), lambda b: (0, b, 0)),
            pl.BlockSpec((BB, H), lambda b: (b, 0)),
            pl.BlockSpec((1, 1, BB), lambda b: (b, 0, 0)),
            pl.BlockSpec((1, 1, BB), lambda b: (b, 0, 0)),
            full((H, NZ)),
            full((H, NZ)),
            full((H, NZ)),
            full((16, 1)),
            full((8, 16)),
            full((8, 1)),
            full((8, 1)),
            full((1, 1)),
        ],
        out_specs=pl.BlockSpec((1, 1, BB), lambda b: (b, 0, 0)),
        out_shape=jax.ShapeDtypeStruct((NB, 1, BB), jnp.float32),
    )(h, q, sl3, bias3, wz_h, wz_m, cq_ext, b1c, w2t, b2c, w3c, bfin)


def kernel(u, i, i_c, hist_i, hist_c, sl, item_table, cat_table, item_bias,
           att_w1, att_b1, att_w2, att_b2, att_w3, att_b3, att_w4, att_b4,
           bn_gamma, bn_beta, fc1_w, fc1_b, fc2_w, fc2_b, fc3_w, fc3_b):
    del u
    w1q, w1h, w1d, w1m = (att_w1[0:H], att_w1[H:2 * H],
                          att_w1[2 * H:3 * H], att_w1[3 * H:4 * H])
    cq = w1q + w1d
    a = w1h - w1d
    g = fc1_w @ fc2_w @ fc3_w
    scale = bn_gamma / jnp.sqrt(1.0 + 1e-3)
    wfin = scale[:, None] * g
    bfin = (bn_beta @ g + (fc1_b @ fc2_w + fc2_b) @ fc3_w + fc3_b
            + att_b4 @ wfin[0:H])
    wp = att_w4 @ wfin[0:H]
    wq = wfin[H:2 * H]

    zpad = jnp.zeros((H, NZ - 17), jnp.float32)
    zcol = jnp.zeros((H, 1), jnp.float32)
    wz_h = jnp.concatenate([a, wp, zpad], axis=1)
    wz_m = jnp.concatenate([w1m, zcol, zpad], axis=1)
    cq_ext = jnp.concatenate([cq, wq, zpad], axis=1)

    h, q, bg = _sc_gather(
        hist_i.T.reshape(-1), hist_c.T.reshape(-1), i, i_c,
        item_table, cat_table, item_bias)

    out = _tc_forward(
        h, q, sl.reshape(NB, 1, BB), bg,
        wz_h, wz_m, cq_ext,
        att_b1.reshape(16, 1), att_w2.T, att_b2.reshape(8, 1),
        att_w3.reshape(8, 1), bfin.reshape(1, 1))
    return out.reshape(-1)
